# baseline (speedup 1.0000x reference)
import jax
import jax.numpy as jnp
from jax.experimental import pallas as pl
from jax.experimental.pallas import tpu as pltpu

_B = 32

_L0 = (66, 200, 3, 31, 98, 24, 5, 2)
_L1 = (31, 98, 24, 14, 47, 36, 5, 2)
_L2 = (14, 47, 36, 5, 22, 48, 5, 2)
_L3 = (5, 22, 48, 3, 20, 64, 3, 1)
_L4 = (3, 20, 64, 1, 18, 64, 3, 1)


def _elu(x):
    return jnp.where(x > 0, x, jnp.exp(jnp.minimum(x, 0.0)) - 1.0)


def _band(h_in, oh, k, stride):
    h = jax.lax.broadcasted_iota(jnp.int32, (h_in, oh, k), 0)
    o = jax.lax.broadcasted_iota(jnp.int32, (h_in, oh, k), 1)
    i = jax.lax.broadcasted_iota(jnp.int32, (h_in, oh, k), 2)
    return (h == stride * o + i).astype(jnp.float32)


def _banded_weights(w, geom, c_major_in=False):
    h_in, _, c_in, oh, _, c_out, k, s = geom
    wr = w.reshape(k, k, c_in, c_out).astype(jnp.float32)
    band = _band(h_in, oh, k, s)
    if c_major_in:
        m = jnp.einsum('hoi,ijck->jchok', band, wr)
        return m.reshape(k, c_in * h_in, oh * c_out).astype(jnp.bfloat16)
    m = jnp.einsum('hoi,ijck->jhcok', band, wr)
    return m.reshape(k, h_in * c_in, oh * c_out).astype(jnp.bfloat16)


def _conv_s2(in_ref, wb_ref, bias, geom, out_ref, p_out):
    _, _, _, oh, ow, c_out, k, _ = geom
    bsz, p_in, _, lanes = in_ref.shape
    for q in range(p_out):
        m_q = (ow - 1 - q) // p_out + 1
        m8 = (m_q + 7) // 8 * 8
        acc = jnp.zeros((bsz * m8, oh * c_out), jnp.float32)
        for j in range(k):
            t = 2 * q + j
            r, st = t % p_in, t // p_in
            slab = in_ref[:, r, st: st + m8, :].reshape(bsz * m8, lanes)
            acc = acc + jnp.dot(slab, wb_ref[j],
                                preferred_element_type=jnp.float32)
        res = _elu(acc + bias).astype(jnp.bfloat16).reshape(
            bsz, m8, oh * c_out)
        if p_out == 1:
            out_ref[:, :m_q, :] = res[:, :m_q, :]
        else:
            out_ref[:, q, :m_q, :] = res[:, :m_q, :]


def _conv_s1(in_ref, wb_ref, bias, geom, out_ref):
    _, _, _, oh, ow, c_out, k, _ = geom
    bsz = in_ref.shape[0]
    lanes = in_ref.shape[2]
    m8 = (ow + 7) // 8 * 8
    acc = jnp.zeros((bsz * m8, oh * c_out), jnp.float32)
    for j in range(k):
        slab = in_ref[:, j: j + m8, :].reshape(bsz * m8, lanes)
        acc = acc + jnp.dot(slab, wb_ref[j],
                            preferred_element_type=jnp.float32)
    res = _elu(acc + bias).astype(jnp.bfloat16).reshape(bsz, m8, oh * c_out)
    out_ref[:, :ow, :] = res[:, :ow, :]


def _fwd_kernel(xp_ref, w0_ref, b0_ref, w1_ref, b1_ref, w2_ref, b2_ref,
                w3_ref, b3_ref, w4_ref, b4_ref,
                wf0_ref, bf0_ref, wf1_ref, bf1_ref, wf2_ref, bf2_ref,
                wf3_ref, bf3_ref, o_ref, a0_s, a1_s, a2_s, a3_s, a4_s,
                f_s):
    _conv_s2(xp_ref, w0_ref, b0_ref[...], _L0, a0_s, 4)
    _conv_s2(a0_s, w1_ref, b1_ref[...], _L1, a1_s, 2)
    _conv_s2(a1_s, w2_ref, b2_ref[...], _L2, a2_s, 1)
    _conv_s1(a2_s, w3_ref, b3_ref[...], _L3, a3_s)
    _conv_s1(a3_s, w4_ref, b4_ref[...], _L4, a4_s)

    for w in range(18):
        f_s[:, 64 * w: 64 * (w + 1)] = a4_s[:, w, :]
    h = _elu(jnp.dot(f_s[...], wf0_ref[...],
                     preferred_element_type=jnp.float32) + bf0_ref[...])
    h = _elu(jnp.dot(h.astype(jnp.bfloat16), wf1_ref[...],
                     preferred_element_type=jnp.float32) + bf1_ref[...])
    h = _elu(jnp.dot(h.astype(jnp.bfloat16), wf2_ref[...],
                     preferred_element_type=jnp.float32) + bf2_ref[...])
    y = jnp.sum(h * wf3_ref[...], axis=-1, keepdims=True) + bf3_ref[...]
    o_ref[...] = jnp.broadcast_to(y, (_B, 128))


def kernel(x, w0, b0, w1, b1, w2, b2, w3, b3, w4, b4,
           wf0, bf0, wf1, bf1, wf2, bf2, wf3, bf3):
    n = x.shape[0]
    xt = jnp.swapaxes(x.reshape(n, 198, 200), 1, 2)
    xp4 = jnp.pad(xt.reshape(n, 25, 8, 198), ((0, 0), (0, 8), (0, 0), (0, 0)))
    xp = jnp.transpose(xp4, (0, 2, 1, 3)).astype(jnp.bfloat16)

    wb0 = _banded_weights(w0, _L0, c_major_in=True)
    wb1 = _banded_weights(w1, _L1)
    wb2 = _banded_weights(w2, _L2)
    wb3 = _banded_weights(w3.reshape(9 * 48, 64), _L3)
    wb4 = _banded_weights(w4, _L4)

    tb0 = jnp.tile(b0, (1, _L0[3]))
    tb1 = jnp.tile(b1, (1, _L1[3]))
    tb2 = jnp.tile(b2, (1, _L2[3]))
    tb3 = jnp.tile(b3, (1, _L3[3]))
    tb4 = jnp.tile(b4, (1, _L4[3]))

    def full(a):
        return pl.BlockSpec(a.shape, lambda i: (0,) * a.ndim)

    weights = (wb0, tb0, wb1, tb1, wb2, tb2, wb3, tb3, wb4, tb4,
               wf0, bf0, wf1, bf1, wf2, bf2, wf3, bf3)
    out = pl.pallas_call(
        _fwd_kernel,
        out_shape=jax.ShapeDtypeStruct((n, 128), jnp.float32),
        grid=(n // _B,),
        in_specs=[pl.BlockSpec((_B, 8, 33, 198), lambda i: (i, 0, 0, 0))]
                 + [full(a) for a in weights],
        out_specs=pl.BlockSpec((_B, 128), lambda i: (i, 0)),
        scratch_shapes=[
            pltpu.VMEM((_B, 4, 25, 744), jnp.bfloat16),
            pltpu.VMEM((_B, 2, 26, 504), jnp.bfloat16),
            pltpu.VMEM((_B, 26, 240), jnp.bfloat16),
            pltpu.VMEM((_B, 26, 192), jnp.bfloat16),
            pltpu.VMEM((_B, 24, 64), jnp.bfloat16),
            pltpu.VMEM((_B, 18 * 64), jnp.bfloat16),
        ],
        compiler_params=pltpu.CompilerParams(dimension_semantics=("parallel",)),
    )(xp, *weights)
    return out[:, :1]


# speedup vs baseline: 22.4625x; 22.4625x over previous
import jax
import jax.numpy as jnp
from jax.experimental import pallas as pl
from jax.experimental.pallas import tpu as pltpu

_B = 32

_L0 = (66, 200, 3, 31, 98, 24, 5, 2)
_L1 = (31, 98, 24, 14, 47, 36, 5, 2)
_L2 = (14, 47, 36, 5, 22, 48, 5, 2)
_L3 = (5, 22, 48, 3, 20, 64, 3, 1)
_L4 = (3, 20, 64, 1, 18, 64, 3, 1)


def _elu(x):
    return jnp.where(x > 0, x, jnp.exp(jnp.minimum(x, 0.0)) - 1.0)


def _band(h_in, oh, k, stride):
    h = jax.lax.broadcasted_iota(jnp.int32, (h_in, oh, k), 0)
    o = jax.lax.broadcasted_iota(jnp.int32, (h_in, oh, k), 1)
    i = jax.lax.broadcasted_iota(jnp.int32, (h_in, oh, k), 2)
    return (h == stride * o + i).astype(jnp.float32)


def _banded_weights(w, geom, c_major_in=False):
    h_in, _, c_in, oh, _, c_out, k, s = geom
    wr = w.reshape(k, k, c_in, c_out).astype(jnp.float32)
    band = _band(h_in, oh, k, s)
    if c_major_in:
        m = jnp.einsum('hoi,ijck->jchok', band, wr)
        return m.reshape(k, c_in * h_in, oh * c_out).astype(jnp.bfloat16)
    m = jnp.einsum('hoi,ijck->jhcok', band, wr)
    return m.reshape(k, h_in * c_in, oh * c_out).astype(jnp.bfloat16)


def _conv_s2(in_ref, wb_ref, bias, geom, out_ref, p_out):
    _, _, _, oh, ow, c_out, k, _ = geom
    bsz, p_in, _, lanes = in_ref.shape
    for q in range(p_out):
        m_q = (ow - 1 - q) // p_out + 1
        m8 = (m_q + 7) // 8 * 8
        acc = jnp.zeros((bsz * m8, oh * c_out), jnp.float32)
        for j in range(k):
            t = 2 * q + j
            r, st = t % p_in, t // p_in
            slab = in_ref[:, r, st: st + m8, :].reshape(bsz * m8, lanes)
            acc = acc + jnp.dot(slab, wb_ref[j],
                                preferred_element_type=jnp.float32)
        res = _elu(acc + bias).astype(jnp.bfloat16).reshape(
            bsz, m8, oh * c_out)
        if p_out == 1:
            out_ref[:, :m_q, :] = res[:, :m_q, :]
        else:
            out_ref[:, q, :m_q, :] = res[:, :m_q, :]


def _conv_s1(in_ref, wb_ref, bias, geom, out_ref):
    _, _, _, oh, ow, c_out, k, _ = geom
    bsz = in_ref.shape[0]
    lanes = in_ref.shape[2]
    m8 = (ow + 7) // 8 * 8
    acc = jnp.zeros((bsz * m8, oh * c_out), jnp.float32)
    for j in range(k):
        slab = in_ref[:, j: j + m8, :].reshape(bsz * m8, lanes)
        acc = acc + jnp.dot(slab, wb_ref[j],
                            preferred_element_type=jnp.float32)
    res = _elu(acc + bias).astype(jnp.bfloat16).reshape(bsz, m8, oh * c_out)
    out_ref[:, :ow, :] = res[:, :ow, :]


def _fwd_kernel(xp_ref, w0_ref, b0_ref, w1_ref, b1_ref, w2_ref, b2_ref,
                w3_ref, b3_ref, w4_ref, b4_ref,
                wf0_ref, bf0_ref, wf1_ref, bf1_ref, wf2_ref, bf2_ref,
                wf3_ref, bf3_ref, o_ref, xb_s, a0_s, a1_s, a2_s, a3_s, a4_s,
                f_s):
    for b in range(_B):
        xb_s[b, :, :25, :] = xp_ref[b].astype(jnp.bfloat16)
        xb_s[b, :, 25:, :] = jnp.zeros((8, 8, 198), jnp.bfloat16)

    _conv_s2(xb_s, w0_ref, b0_ref[...], _L0, a0_s, 4)
    _conv_s2(a0_s, w1_ref, b1_ref[...], _L1, a1_s, 2)
    _conv_s2(a1_s, w2_ref, b2_ref[...], _L2, a2_s, 1)
    _conv_s1(a2_s, w3_ref, b3_ref[...], _L3, a3_s)
    _conv_s1(a3_s, w4_ref, b4_ref[...], _L4, a4_s)

    for w in range(18):
        f_s[:, 64 * w: 64 * (w + 1)] = a4_s[:, w, :]
    h = _elu(jnp.dot(f_s[...], wf0_ref[...],
                     preferred_element_type=jnp.float32) + bf0_ref[...])
    h = _elu(jnp.dot(h.astype(jnp.bfloat16), wf1_ref[...],
                     preferred_element_type=jnp.float32) + bf1_ref[...])
    h = _elu(jnp.dot(h.astype(jnp.bfloat16), wf2_ref[...],
                     preferred_element_type=jnp.float32) + bf2_ref[...])
    y = jnp.sum(h * wf3_ref[...], axis=-1, keepdims=True) + bf3_ref[...]
    o_ref[...] = jnp.broadcast_to(y, (_B, 128))


def kernel(x, w0, b0, w1, b1, w2, b2, w3, b3, w4, b4,
           wf0, bf0, wf1, bf1, wf2, bf2, wf3, bf3):
    n = x.shape[0]
    xt = jnp.swapaxes(x.reshape(n, 198, 200), 1, 2)
    xp = jnp.transpose(xt.reshape(n, 25, 8, 198), (0, 2, 1, 3))

    wb0 = _banded_weights(w0, _L0, c_major_in=True)
    wb1 = _banded_weights(w1, _L1)
    wb2 = _banded_weights(w2, _L2)
    wb3 = _banded_weights(w3.reshape(9 * 48, 64), _L3)
    wb4 = _banded_weights(w4, _L4)

    tb0 = jnp.tile(b0, (1, _L0[3]))
    tb1 = jnp.tile(b1, (1, _L1[3]))
    tb2 = jnp.tile(b2, (1, _L2[3]))
    tb3 = jnp.tile(b3, (1, _L3[3]))
    tb4 = jnp.tile(b4, (1, _L4[3]))

    def full(a):
        return pl.BlockSpec(a.shape, lambda i: (0,) * a.ndim)

    weights = (wb0, tb0, wb1, tb1, wb2, tb2, wb3, tb3, wb4, tb4,
               wf0, bf0, wf1, bf1, wf2, bf2, wf3, bf3)
    out = pl.pallas_call(
        _fwd_kernel,
        out_shape=jax.ShapeDtypeStruct((n, 128), jnp.float32),
        grid=(n // _B,),
        in_specs=[pl.BlockSpec((_B, 8, 25, 198), lambda i: (i, 0, 0, 0))]
                 + [full(a) for a in weights],
        out_specs=pl.BlockSpec((_B, 128), lambda i: (i, 0)),
        scratch_shapes=[
            pltpu.VMEM((_B, 8, 33, 198), jnp.bfloat16),
            pltpu.VMEM((_B, 4, 25, 744), jnp.bfloat16),
            pltpu.VMEM((_B, 2, 26, 504), jnp.bfloat16),
            pltpu.VMEM((_B, 26, 240), jnp.bfloat16),
            pltpu.VMEM((_B, 26, 192), jnp.bfloat16),
            pltpu.VMEM((_B, 24, 64), jnp.bfloat16),
            pltpu.VMEM((_B, 18 * 64), jnp.bfloat16),
        ],
        compiler_params=pltpu.CompilerParams(dimension_semantics=("parallel",)),
    )(xp, *weights)
    return out[:, :1]
